# Initial kernel scaffold; baseline (speedup 1.0000x reference)
import jax
import jax.numpy as jnp
from jax.experimental import pallas as pl


def kernel(mel, W, b, codebooks):
    raise NotImplementedError("write your pallas kernel here")


# speedup vs baseline: 5.4414x; 5.4414x over previous
import jax
import jax.numpy as jnp
from jax import lax
from jax.experimental import pallas as pl
from jax.experimental.pallas import tpu as pltpu

_B = 16
_D = 64
_T = 48
_K = 8192
_NCB = 4

_NT = (((1,), (1,)), ((), ()))


def _rvq_body(mel_ref, w0_ref, w1_ref, b_ref, cb_ref, cbn_ref,
              q_ref, idx_ref, commit_ref, util_ref):
    b_row = b_ref[0]

    def step(t, carry):
        prev1, prev2, commit_acc, util_acc = carry
        a0 = lax.dot_general(prev2, w0_ref[...], _NT,
                             preferred_element_type=jnp.float32)
        a1 = lax.dot_general(prev1, w1_ref[...], _NT,
                             preferred_element_type=jnp.float32)
        pred = (a0 + a1) + b_row
        pred = jnp.where(t == 0, jnp.zeros_like(pred), pred)

        x = mel_ref[t]
        resid0 = x - pred
        resid = resid0
        quant = jnp.zeros_like(resid)
        commit_f = jnp.zeros((1, 1), jnp.float32)
        util_f = jnp.zeros((1, 1), jnp.float32)
        iota_k = lax.broadcasted_iota(jnp.int32, (_B, _K), 1)

        for c in range(_NCB):
            cb = cb_ref[c]
            rss = jnp.sum(resid * resid, axis=1, keepdims=True)
            prod = lax.dot_general(resid, cb, _NT,
                                   preferred_element_type=jnp.float32)
            d = (rss - 2.0 * prod) + cbn_ref[c]
            dmin = jnp.min(d, axis=1, keepdims=True)
            idx = jnp.min(jnp.where(d == dmin, iota_k, _K),
                          axis=1, keepdims=True)
            onehot = (iota_k == idx).astype(jnp.float32)
            q = lax.dot_general(onehot, cb, (((1,), (0,)), ((), ())),
                                precision=lax.Precision.HIGHEST,
                                preferred_element_type=jnp.float32)
            diff = resid - q
            commit_f = commit_f + jnp.sum(
                jnp.sum(diff * diff, axis=1, keepdims=True),
                axis=0, keepdims=True) * (1.0 / (_B * _D))
            presence = jnp.max(onehot, axis=0, keepdims=True)
            util_f = util_f + (jnp.sum(presence, axis=1, keepdims=True)
                               / float(_K))
            quant = quant + q
            resid = resid - q
            idx_ref[t * _NCB + c] = idx

        quant_st = resid0 + (quant - resid0)
        mel_t = pred + quant_st
        q_ref[t] = mel_t
        return (mel_t, prev1,
                commit_acc + commit_f,
                util_acc + util_f * 0.25)

    zeros = mel_ref[0] * 0.0
    zacc = jnp.zeros((1, 1), jnp.float32)
    _, _, commit_acc, util_acc = lax.fori_loop(
        0, _T, step, (zeros, zeros, zacc, zacc))
    commit_ref[...] = commit_acc
    util_ref[...] = util_acc


def kernel(mel, W, b, codebooks):
    mel_t = mel.transpose(2, 0, 1)
    w0 = W[:, :, 0]
    w1 = W[:, :, 1]
    b2 = b.reshape(1, 1, _D)
    cbn = jnp.sum(codebooks ** 2, axis=2).reshape(_NCB, 1, _K)

    q_out, idx_out, commit, util = pl.pallas_call(
        _rvq_body,
        out_shape=(
            jax.ShapeDtypeStruct((_T, _B, _D), jnp.float32),
            jax.ShapeDtypeStruct((_T * _NCB, _B, 1), jnp.int32),
            jax.ShapeDtypeStruct((1, 1), jnp.float32),
            jax.ShapeDtypeStruct((1, 1), jnp.float32),
        ),
    )(mel_t, w0, w1, b2, codebooks, cbn)

    mel_q = q_out.transpose(1, 2, 0)
    all_idx = idx_out.reshape(_T, _NCB, _B)
    return mel_q, all_idx, commit[0, 0] / _T, util[0, 0] / _T


# speedup vs baseline: 13.2826x; 2.4410x over previous
import jax
import jax.numpy as jnp
from jax import lax
from jax.experimental import pallas as pl

_B = 16
_D = 64
_T = 48
_K = 8192
_NCB = 4

_NT = (((1,), (1,)), ((), ()))


def _rvq_body(mel_ref, w0_ref, w1_ref, b_ref, cb_ref, cbn_ref,
              q_ref, idx_ref, commit_ref, util_ref):
    b_row = b_ref[0]
    iota_k = lax.broadcasted_iota(jnp.int32, (_B, _K), 1)
    riota = lax.broadcasted_iota(jnp.int32, (_B, 1), 0)
    ciota = lax.broadcasted_iota(jnp.int32, (1, _B), 1)
    tri = (lax.broadcasted_iota(jnp.int32, (_B, _B), 1)
           < lax.broadcasted_iota(jnp.int32, (_B, _B), 0))

    def step(t, carry):
        prev1, prev2, commit_acc, util_acc = carry
        a0 = lax.dot_general(prev2, w0_ref[...], _NT,
                             preferred_element_type=jnp.float32)
        a1 = lax.dot_general(prev1, w1_ref[...], _NT,
                             preferred_element_type=jnp.float32)
        pred = (a0 + a1) + b_row
        pred = jnp.where(t == 0, jnp.zeros_like(pred), pred)

        x = mel_ref[t]
        resid0 = x - pred
        resid = resid0
        quant = jnp.zeros_like(resid)
        commit_f = jnp.zeros((1, 1), jnp.float32)
        util_f = jnp.zeros((1, 1), jnp.float32)

        for c in range(_NCB):
            cb = cb_ref[c]
            rss = jnp.sum(resid * resid, axis=1, keepdims=True)
            prod = lax.dot_general(resid, cb, _NT,
                                   preferred_element_type=jnp.float32)
            d = (rss - 2.0 * prod) + cbn_ref[c]
            dmin = jnp.min(d, axis=1, keepdims=True)
            idx = jnp.min(jnp.where(d == dmin, iota_k, _K),
                          axis=1, keepdims=True)
            rows = []
            idx_row = jnp.zeros((1, _B), jnp.int32)
            for bb in range(_B):
                sel = jnp.sum(jnp.where(riota == bb, idx, 0))
                idx_row = jnp.where(ciota == bb, sel, idx_row)
                rows.append(cb_ref[c, pl.ds(sel, 1), :])
            q = jnp.concatenate(rows, axis=0)
            diff = resid - q
            commit_f = commit_f + jnp.sum(
                jnp.sum(diff * diff, axis=1, keepdims=True),
                axis=0, keepdims=True) * (1.0 / (_B * _D))
            em = jnp.broadcast_to(idx_row, (_B, _B))
            dup = jnp.any((em == idx) & tri, axis=1, keepdims=True)
            uniq = jnp.sum(jnp.where(dup, 0.0, 1.0), axis=0,
                           keepdims=True)
            util_f = util_f + uniq / float(_K)
            quant = quant + q
            resid = diff
            idx_ref[t * _NCB + c] = idx

        quant_st = resid0 + (quant - resid0)
        mel_t = pred + quant_st
        q_ref[t] = mel_t
        return (mel_t, prev1,
                commit_acc + commit_f,
                util_acc + util_f * 0.25)

    zeros = mel_ref[0] * 0.0
    zacc = jnp.zeros((1, 1), jnp.float32)
    _, _, commit_acc, util_acc = lax.fori_loop(
        0, _T, step, (zeros, zeros, zacc, zacc))
    commit_ref[...] = commit_acc
    util_ref[...] = util_acc


def kernel(mel, W, b, codebooks):
    mel_t = mel.transpose(2, 0, 1)
    w0 = W[:, :, 0]
    w1 = W[:, :, 1]
    b2 = b.reshape(1, 1, _D)
    cbn = jnp.sum(codebooks ** 2, axis=2).reshape(_NCB, 1, _K)

    q_out, idx_out, commit, util = pl.pallas_call(
        _rvq_body,
        out_shape=(
            jax.ShapeDtypeStruct((_T, _B, _D), jnp.float32),
            jax.ShapeDtypeStruct((_T * _NCB, _B, 1), jnp.int32),
            jax.ShapeDtypeStruct((1, 1), jnp.float32),
            jax.ShapeDtypeStruct((1, 1), jnp.float32),
        ),
    )(mel_t, w0, w1, b2, codebooks, cbn)

    mel_q = q_out.transpose(1, 2, 0)
    all_idx = idx_out.reshape(_T, _NCB, _B)
    return mel_q, all_idx, commit[0, 0] / _T, util[0, 0] / _T


# speedup vs baseline: 13.7737x; 1.0370x over previous
import jax
import jax.numpy as jnp
from jax import lax
from jax.experimental import pallas as pl

_B = 16
_D = 64
_T = 48
_K = 8192
_NCB = 4
_KT = 8
_TK = _K // _KT

_NT = (((1,), (1,)), ((), ()))


def _rvq_body(mel_ref, w0_ref, w1_ref, b_ref, cb_ref, cbn_ref,
              q_ref, idx_ref, commit_ref, util_ref):
    b_row = b_ref[0]
    iota_t = lax.broadcasted_iota(jnp.int32, (_B, _TK), 1)
    riota = lax.broadcasted_iota(jnp.int32, (_B, 1), 0)
    ciota = lax.broadcasted_iota(jnp.int32, (1, _B), 1)
    tri = (lax.broadcasted_iota(jnp.int32, (_B, _B), 1)
           < lax.broadcasted_iota(jnp.int32, (_B, _B), 0))

    def step(t, carry):
        prev1, prev2, commit_acc, util_acc = carry
        a0 = lax.dot_general(prev2, w0_ref[...], _NT,
                             preferred_element_type=jnp.float32)
        a1 = lax.dot_general(prev1, w1_ref[...], _NT,
                             preferred_element_type=jnp.float32)
        pred = (a0 + a1) + b_row
        pred = jnp.where(t == 0, jnp.zeros_like(pred), pred)

        x = mel_ref[t]
        resid0 = x - pred
        resid = resid0
        quant = jnp.zeros_like(resid)
        commit_f = jnp.zeros((1, 1), jnp.float32)
        util_f = jnp.zeros((1, 1), jnp.float32)

        for c in range(_NCB):
            rss = jnp.sum(resid * resid, axis=1, keepdims=True)
            m_run = None
            i_run = None
            for kt in range(_KT):
                sl = pl.ds(kt * _TK, _TK)
                prod = lax.dot_general(resid, cb_ref[c, sl, :], _NT,
                                       preferred_element_type=jnp.float32)
                d = (rss - 2.0 * prod) + cbn_ref[c, :, sl]
                m_t = jnp.min(d, axis=1, keepdims=True)
                i_t = jnp.min(jnp.where(d == m_t, iota_t + (kt * _TK), _K),
                              axis=1, keepdims=True)
                if m_run is None:
                    m_run, i_run = m_t, i_t
                else:
                    i_run = jnp.where(m_t < m_run, i_t, i_run)
                    m_run = jnp.minimum(m_run, m_t)
            idx = i_run
            rows = []
            idx_row = jnp.zeros((1, _B), jnp.int32)
            for bb in range(_B):
                sel = jnp.sum(jnp.where(riota == bb, idx, 0))
                idx_row = jnp.where(ciota == bb, sel, idx_row)
                rows.append(cb_ref[c, pl.ds(sel, 1), :])
            q = jnp.concatenate(rows, axis=0)
            diff = resid - q
            commit_f = commit_f + jnp.sum(
                jnp.sum(diff * diff, axis=1, keepdims=True),
                axis=0, keepdims=True) * (1.0 / (_B * _D))
            em = jnp.broadcast_to(idx_row, (_B, _B))
            dup = jnp.any((em == idx) & tri, axis=1, keepdims=True)
            uniq = jnp.sum(jnp.where(dup, 0.0, 1.0), axis=0,
                           keepdims=True)
            util_f = util_f + uniq / float(_K)
            quant = quant + q
            resid = diff
            idx_ref[t * _NCB + c] = idx

        quant_st = resid0 + (quant - resid0)
        mel_t = pred + quant_st
        q_ref[t] = mel_t
        return (mel_t, prev1,
                commit_acc + commit_f,
                util_acc + util_f * 0.25)

    zeros = mel_ref[0] * 0.0
    zacc = jnp.zeros((1, 1), jnp.float32)
    _, _, commit_acc, util_acc = lax.fori_loop(
        0, _T, step, (zeros, zeros, zacc, zacc))
    commit_ref[...] = commit_acc
    util_ref[...] = util_acc


def kernel(mel, W, b, codebooks):
    mel_t = mel.transpose(2, 0, 1)
    w0 = W[:, :, 0]
    w1 = W[:, :, 1]
    b2 = b.reshape(1, 1, _D)
    cbn = jnp.sum(codebooks ** 2, axis=2).reshape(_NCB, 1, _K)

    q_out, idx_out, commit, util = pl.pallas_call(
        _rvq_body,
        out_shape=(
            jax.ShapeDtypeStruct((_T, _B, _D), jnp.float32),
            jax.ShapeDtypeStruct((_T * _NCB, _B, 1), jnp.int32),
            jax.ShapeDtypeStruct((1, 1), jnp.float32),
            jax.ShapeDtypeStruct((1, 1), jnp.float32),
        ),
    )(mel_t, w0, w1, b2, codebooks, cbn)

    mel_q = q_out.transpose(1, 2, 0)
    all_idx = idx_out.reshape(_T, _NCB, _B)
    return mel_q, all_idx, commit[0, 0] / _T, util[0, 0] / _T


# speedup vs baseline: 13.8918x; 1.0086x over previous
import jax
import jax.numpy as jnp
from jax import lax
from jax.experimental import pallas as pl

_B = 16
_D = 64
_T = 48
_K = 8192
_NCB = 4
_KT = 8
_TK = _K // _KT

_NT = (((1,), (1,)), ((), ()))


def _rvq_body(mel_ref, w0_ref, w1_ref, b_ref, cb_ref, cbn_ref,
              q_ref, idx_ref, commit_ref, util_ref):
    b_row = b_ref[0]
    iota_t = lax.broadcasted_iota(jnp.int32, (_B, _TK), 1)
    riota = lax.broadcasted_iota(jnp.int32, (_B, 1), 0)
    ciota = lax.broadcasted_iota(jnp.int32, (1, _B), 1)
    tri = (lax.broadcasted_iota(jnp.int32, (_B, _B), 1)
           < lax.broadcasted_iota(jnp.int32, (_B, _B), 0))

    def step(t, carry):
        prev1, prev2, commit_acc, util_acc = carry
        a0 = lax.dot_general(prev2, w0_ref[...], _NT,
                             preferred_element_type=jnp.float32)
        a1 = lax.dot_general(prev1, w1_ref[...], _NT,
                             preferred_element_type=jnp.float32)
        pred = (a0 + a1) + b_row
        pred = jnp.where(t == 0, jnp.zeros_like(pred), pred)

        x = mel_ref[t]
        resid0 = x - pred
        resid = resid0
        quant = jnp.zeros_like(resid)
        commit_f = jnp.zeros((1, 1), jnp.float32)
        util_f = jnp.zeros((1, 1), jnp.float32)

        for c in range(_NCB):
            rss = jnp.sum(resid * resid, axis=1, keepdims=True)
            m_run = None
            i_run = None
            for kt in range(_KT):
                sl = pl.ds(kt * _TK, _TK)
                prod = lax.dot_general(resid, cb_ref[c, sl, :], _NT,
                                       preferred_element_type=jnp.float32)
                d = (rss - 2.0 * prod) + cbn_ref[c, :, sl]
                m_t = jnp.min(d, axis=1, keepdims=True)
                i_t = jnp.min(jnp.where(d == m_t, iota_t + (kt * _TK), _K),
                              axis=1, keepdims=True)
                if m_run is None:
                    m_run, i_run = m_t, i_t
                else:
                    i_run = jnp.where(m_t < m_run, i_t, i_run)
                    m_run = jnp.minimum(m_run, m_t)
            idx = i_run
            rows = []
            idx_row = jnp.zeros((1, _B), jnp.int32)
            for bb in range(_B):
                sel = jnp.sum(jnp.where(riota == bb, idx, 0))
                idx_row = jnp.where(ciota == bb, sel, idx_row)
                rows.append(cb_ref[c, pl.ds(sel, 1), :])
            q = jnp.concatenate(rows, axis=0)
            diff = resid - q
            commit_f = commit_f + jnp.sum(
                jnp.sum(diff * diff, axis=1, keepdims=True),
                axis=0, keepdims=True) * (1.0 / (_B * _D))
            em = jnp.broadcast_to(idx_row, (_B, _B))
            dup = jnp.any((em == idx) & tri, axis=1, keepdims=True)
            uniq = jnp.sum(jnp.where(dup, 0.0, 1.0), axis=0,
                           keepdims=True)
            util_f = util_f + uniq / float(_K)
            quant = quant + q
            resid = diff
            idx_ref[t * _NCB + c] = idx

        quant_st = resid0 + (quant - resid0)
        mel_t = pred + quant_st
        q_ref[t] = mel_t
        return (mel_t, prev1,
                commit_acc + commit_f,
                util_acc + util_f * 0.25)

    zeros = mel_ref[0] * 0.0
    zacc = jnp.zeros((1, 1), jnp.float32)
    _, _, commit_acc, util_acc = lax.fori_loop(
        0, _T, step, (zeros, zeros, zacc, zacc), unroll=2)
    commit_ref[...] = commit_acc
    util_ref[...] = util_acc


def kernel(mel, W, b, codebooks):
    mel_t = mel.transpose(2, 0, 1)
    w0 = W[:, :, 0]
    w1 = W[:, :, 1]
    b2 = b.reshape(1, 1, _D)
    cbn = jnp.sum(codebooks ** 2, axis=2).reshape(_NCB, 1, _K)

    q_out, idx_out, commit, util = pl.pallas_call(
        _rvq_body,
        out_shape=(
            jax.ShapeDtypeStruct((_T, _B, _D), jnp.float32),
            jax.ShapeDtypeStruct((_T * _NCB, _B, 1), jnp.int32),
            jax.ShapeDtypeStruct((1, 1), jnp.float32),
            jax.ShapeDtypeStruct((1, 1), jnp.float32),
        ),
    )(mel_t, w0, w1, b2, codebooks, cbn)

    mel_q = q_out.transpose(1, 2, 0)
    all_idx = idx_out.reshape(_T, _NCB, _B)
    return mel_q, all_idx, commit[0, 0] / _T, util[0, 0] / _T


# speedup vs baseline: 13.9559x; 1.0046x over previous
import jax
import jax.numpy as jnp
from jax import lax
from jax.experimental import pallas as pl

_B = 16
_D = 64
_T = 48
_K = 8192
_NCB = 4
_KT = 8
_TK = _K // _KT

_NT = (((1,), (1,)), ((), ()))


def _rvq_body(mel_ref, w0_ref, w1_ref, b_ref, cb_ref, cbn_ref,
              q_ref, idx_ref, commit_ref, util_ref):
    b_row = b_ref[0]
    iota_t = lax.broadcasted_iota(jnp.int32, (_B, _TK), 1)
    riota = lax.broadcasted_iota(jnp.int32, (_B, 1), 0)
    ciota = lax.broadcasted_iota(jnp.int32, (1, _B), 1)
    tri = (lax.broadcasted_iota(jnp.int32, (_B, _B), 1)
           < lax.broadcasted_iota(jnp.int32, (_B, _B), 0))

    def step(t, carry):
        prev1, prev2, commit_acc, util_acc = carry
        a0 = lax.dot_general(prev2, w0_ref[...], _NT,
                             preferred_element_type=jnp.float32)
        a1 = lax.dot_general(prev1, w1_ref[...], _NT,
                             preferred_element_type=jnp.float32)
        pred = (a0 + a1) + b_row
        pred = jnp.where(t == 0, jnp.zeros_like(pred), pred)

        x = mel_ref[t]
        resid0 = x - pred
        resid = resid0
        quant = jnp.zeros_like(resid)
        commit_f = jnp.zeros((1, 1), jnp.float32)
        util_f = jnp.zeros((1, 1), jnp.float32)

        for c in range(_NCB):
            rss = jnp.sum(resid * resid, axis=1, keepdims=True)
            m_run = None
            i_run = None
            for kt in range(_KT):
                sl = pl.ds(kt * _TK, _TK)
                prod = lax.dot_general(resid, cb_ref[c, sl, :], _NT,
                                       preferred_element_type=jnp.float32)
                d = (rss - 2.0 * prod) + cbn_ref[c, :, sl]
                m_t = jnp.min(d, axis=1, keepdims=True)
                i_t = jnp.min(jnp.where(d == m_t, iota_t + (kt * _TK), _K),
                              axis=1, keepdims=True)
                if m_run is None:
                    m_run, i_run = m_t, i_t
                else:
                    i_run = jnp.where(m_t < m_run, i_t, i_run)
                    m_run = jnp.minimum(m_run, m_t)
            idx = i_run
            rows = []
            idx_row = jnp.zeros((1, _B), jnp.int32)
            for bb in range(_B):
                sel = jnp.sum(jnp.where(riota == bb, idx, 0))
                idx_row = jnp.where(ciota == bb, sel, idx_row)
                rows.append(cb_ref[c, pl.ds(sel, 1), :])
            q = jnp.concatenate(rows, axis=0)
            diff = resid - q
            commit_f = commit_f + jnp.sum(
                jnp.sum(diff * diff, axis=1, keepdims=True),
                axis=0, keepdims=True) * (1.0 / (_B * _D))
            em = jnp.broadcast_to(idx_row, (_B, _B))
            dup = jnp.any((em == idx) & tri, axis=1, keepdims=True)
            uniq = jnp.sum(jnp.where(dup, 0.0, 1.0), axis=0,
                           keepdims=True)
            util_f = util_f + uniq / float(_K)
            quant = quant + q
            resid = diff
            idx_ref[t * _NCB + c] = idx

        quant_st = resid0 + (quant - resid0)
        mel_t = pred + quant_st
        q_ref[t] = mel_t
        return (mel_t, prev1,
                commit_acc + commit_f,
                util_acc + util_f * 0.25)

    zeros = mel_ref[0] * 0.0
    zacc = jnp.zeros((1, 1), jnp.float32)
    _, _, commit_acc, util_acc = lax.fori_loop(
        0, _T, step, (zeros, zeros, zacc, zacc), unroll=4)
    commit_ref[...] = commit_acc
    util_ref[...] = util_acc


def kernel(mel, W, b, codebooks):
    mel_t = mel.transpose(2, 0, 1)
    w0 = W[:, :, 0]
    w1 = W[:, :, 1]
    b2 = b.reshape(1, 1, _D)
    cbn = jnp.sum(codebooks ** 2, axis=2).reshape(_NCB, 1, _K)

    q_out, idx_out, commit, util = pl.pallas_call(
        _rvq_body,
        out_shape=(
            jax.ShapeDtypeStruct((_T, _B, _D), jnp.float32),
            jax.ShapeDtypeStruct((_T * _NCB, _B, 1), jnp.int32),
            jax.ShapeDtypeStruct((1, 1), jnp.float32),
            jax.ShapeDtypeStruct((1, 1), jnp.float32),
        ),
    )(mel_t, w0, w1, b2, codebooks, cbn)

    mel_q = q_out.transpose(1, 2, 0)
    all_idx = idx_out.reshape(_T, _NCB, _B)
    return mel_q, all_idx, commit[0, 0] / _T, util[0, 0] / _T
